# baseline (speedup 1.0000x reference)
import functools

import jax
import jax.numpy as jnp
from jax import lax
from jax.experimental import pallas as pl
from jax.experimental.pallas import tpu as pltpu
from jax.experimental.pallas import tpu_sc as plsc

N_TAG = 1000
N_QUES = 1000000
EMB_DIM = 64
BATCH = 16384

_SC_INFO = plsc.get_sparse_core_info()
_NC = _SC_INFO.num_cores
_NS = _SC_INFO.num_subcores
_NW = _NC * _NS
_BPW = BATCH // _NW


_GCH = 128


_PCW = 8192
_Q = 1 << 18


def _sc_gather_body(qp_hbm, ids_hbm, out_hbm, idx_v, pidx_v, pairs_v, sem):
    wid = lax.axis_index("s") * _NC + lax.axis_index("c")
    base = wid * _BPW

    pltpu.sync_copy(ids_hbm.at[pl.ds(base, _BPW)], idx_v)

    for c in range(_BPW // _GCH):
        def _pidx(g):
            rel = idx_v[pl.ds(c * _GCH + g * 16, 16)] - N_TAG
            pidx_v[pl.ds(g * 16, 16)] = rel & (_Q - 1)

        pl.loop(0, _GCH // 16)(_pidx)

        pltpu.async_copy(qp_hbm.at[pidx_v], pairs_v, sem).wait()
        pltpu.sync_copy(pairs_v, out_hbm.at[pl.ds(base + c * _GCH, _GCH)])


_sc_gather = functools.partial(
    pl.kernel,
    mesh=plsc.VectorSubcoreMesh(core_axis_name="c", subcore_axis_name="s"),
    out_type=jax.ShapeDtypeStruct((BATCH, 2 * EMB_DIM), jnp.int32),
    scratch_types=[
        pltpu.VMEM((_BPW,), jnp.int32),
        pltpu.VMEM((_GCH,), jnp.int32),
        pltpu.VMEM((_GCH, 2 * EMB_DIM), jnp.int32),
        pltpu.SemaphoreType.DMA,
    ],
)(_sc_gather_body)


_COLS = 1024
_N_TILES = BATCH // _COLS

_W = EMB_DIM // 2


def _tc_pack_body(a_ref, b_ref, c_ref, d_ref, o_ref):
    eye = (lax.broadcasted_iota(jnp.int32, (EMB_DIM, EMB_DIM), 0)
           == lax.broadcasted_iota(jnp.int32, (EMB_DIM, EMB_DIM), 1)
           ).astype(jnp.float32)

    def tr_pack(ref):
        t = lax.dot_general(ref[...], eye, (((0,), (0,)), ((), ())),
                            preferred_element_type=jnp.float32)
        xb = lax.bitcast_convert_type(t, jnp.int32)
        rb = lax.shift_right_logical(
            xb + 0x7FFF + (lax.shift_right_logical(xb, 16) & 1), 16)
        return rb[:, :_W] | lax.shift_left(rb[:, _W:], 16)

    for q, ref in enumerate((a_ref, b_ref, c_ref, d_ref)):
        o_ref[:, q * _W:(q + 1) * _W] = tr_pack(ref)


def _tc_pack(qt):
    n_steps = _Q // _PCW
    qb = _Q // _PCW

    def _im(q):
        return lambda i: (0, jnp.minimum(q * qb + i, N_QUES // _PCW))

    return pl.pallas_call(
        _tc_pack_body,
        grid=(n_steps,),
        in_specs=[pl.BlockSpec((EMB_DIM, _PCW), _im(q)) for q in range(4)],
        out_specs=pl.BlockSpec((_PCW, 4 * _W), lambda i: (i, 0)),
        out_shape=jax.ShapeDtypeStruct((_Q, 4 * _W), jnp.int32),
    )(qt, qt, qt, qt)


def _softmax_cols(logits):
    m = jnp.max(logits, axis=0, keepdims=True)
    e = jnp.exp(logits - m)
    return e / jnp.sum(e, axis=0, keepdims=True)


def _tc_tag_body(ids_ref, tagt_ref, fwt_ref, b_ref, o_ref):
    ids = ids_ref[0]
    tgrid = lax.broadcasted_iota(jnp.int32, (N_TAG, _COLS), 0)
    onehot = (tgrid == ids).astype(jnp.float32)
    zt = lax.dot_general(tagt_ref[...], onehot, (((1,), (0,)), ((), ())),
                         preferred_element_type=jnp.float32)
    logits = lax.dot_general(fwt_ref[...], zt, (((0,), (0,)), ((), ())),
                             preferred_element_type=jnp.float32)
    o_ref[...] = _softmax_cols(logits + b_ref[...])


def _tc_ques_body(zp_ref, qid_ref, fwt_ref, b_ref, _alias_ref, o_ref):
    zp = zp_ref[...]
    q = lax.shift_right_logical(qid_ref[0] - N_TAG, 18)
    w01 = jnp.where((q & 1) == 1, zp[:, _W:2 * _W], zp[:, :_W])
    w23 = jnp.where((q & 1) == 1, zp[:, 3 * _W:], zp[:, 2 * _W:3 * _W])
    w = jnp.where(q >= 2, w23, w01)
    lo = lax.bitcast_convert_type(lax.shift_left(w, 16), jnp.float32)
    hi = lax.bitcast_convert_type(w & jnp.int32(-65536), jnp.float32)
    z = jnp.concatenate([lo, hi], axis=1)
    logits = lax.dot_general(fwt_ref[...], z, (((0,), (1,)), ((), ())),
                             preferred_element_type=jnp.float32)
    o_ref[...] = _softmax_cols(logits + b_ref[...])


def _tc_tag(ids3, tagt, fwt, b2):
    return pl.pallas_call(
        _tc_tag_body,
        grid=(_N_TILES,),
        in_specs=[
            pl.BlockSpec((1, 1, _COLS), lambda i: (i, 0, 0)),
            pl.BlockSpec((EMB_DIM, N_TAG), lambda i: (0, 0)),
            pl.BlockSpec((EMB_DIM, N_TAG), lambda i: (0, 0)),
            pl.BlockSpec((N_TAG, 1), lambda i: (0, 0)),
        ],
        out_specs=pl.BlockSpec((N_TAG, _COLS), lambda i: (0, i)),
        out_shape=jax.ShapeDtypeStruct((N_TAG, 2 * BATCH), jnp.float32),
    )(ids3, tagt, fwt, b2)


def _tc_ques(zp, qid3, fwt, b2, out_buf):
    return pl.pallas_call(
        _tc_ques_body,
        grid=(_N_TILES,),
        in_specs=[
            pl.BlockSpec((_COLS, 4 * _W), lambda i: (i, 0)),
            pl.BlockSpec((1, _COLS, 1), lambda i: (i, 0, 0)),
            pl.BlockSpec((EMB_DIM, N_TAG), lambda i: (0, 0)),
            pl.BlockSpec((N_TAG, 1), lambda i: (0, 0)),
            pl.BlockSpec(memory_space=pl.ANY),
        ],
        out_specs=pl.BlockSpec((N_TAG, _COLS), lambda i: (0, _N_TILES + i)),
        out_shape=jax.ShapeDtypeStruct((N_TAG, 2 * BATCH), jnp.float32),
        input_output_aliases={4: 0},
    )(zp, qid3, fwt, b2, out_buf)


def kernel(tag_ids, ques_ids, tag_table, ques_table, fc_w, fc_b):
    tag_ids = tag_ids.astype(jnp.int32)
    ques_ids = ques_ids.astype(jnp.int32)
    qcat = _tc_pack(ques_table.T)
    zp = _sc_gather(qcat, ques_ids)
    tagt = tag_table.T
    fwt = fc_w.T
    b2 = fc_b.reshape(N_TAG, 1)
    ids3 = tag_ids.reshape(_N_TILES, 1, _COLS)
    qid3 = ques_ids.reshape(_N_TILES, _COLS, 1)
    out_t = _tc_tag(ids3, tagt, fwt, b2)
    out_t = _tc_ques(zp, qid3, fwt, b2, out_t)
    return out_t.T


# speedup vs baseline: 1.3948x; 1.3948x over previous
import functools

import jax
import jax.numpy as jnp
from jax import lax
from jax.experimental import pallas as pl
from jax.experimental.pallas import tpu as pltpu
from jax.experimental.pallas import tpu_sc as plsc

N_TAG = 1000
N_QUES = 1000000
EMB_DIM = 64
BATCH = 16384

_SC_INFO = plsc.get_sparse_core_info()
_NC = _SC_INFO.num_cores
_NS = _SC_INFO.num_subcores
_NW = _NC * _NS
_BPW = BATCH // _NW


_GCH = 128


_PCW = 16384
_HALF = 31 * _PCW


def _sc_gather_body(qp_hbm, ids_hbm, out_hbm, idx_v, pidx_v, pairs_v, sem):
    wid = lax.axis_index("s") * _NC + lax.axis_index("c")
    base = wid * _BPW

    pltpu.sync_copy(ids_hbm.at[pl.ds(base, _BPW)], idx_v)

    for c in range(_BPW // _GCH):
        def _pidx(g):
            rel = idx_v[pl.ds(c * _GCH + g * 16, 16)] - N_TAG
            wrap = jnp.where(rel >= _HALF, rel - _HALF, rel)
            pidx_v[pl.ds(g * 16, 16)] = wrap

        pl.loop(0, _GCH // 16)(_pidx)

        pltpu.async_copy(qp_hbm.at[pidx_v], pairs_v, sem).wait()
        pltpu.sync_copy(pairs_v, out_hbm.at[pl.ds(base + c * _GCH, _GCH)])


_sc_gather = functools.partial(
    pl.kernel,
    mesh=plsc.VectorSubcoreMesh(core_axis_name="c", subcore_axis_name="s"),
    out_type=jax.ShapeDtypeStruct((BATCH, 2 * EMB_DIM), jnp.float32),
    scratch_types=[
        pltpu.VMEM((_BPW,), jnp.int32),
        pltpu.VMEM((_GCH,), jnp.int32),
        pltpu.VMEM((_GCH, 2 * EMB_DIM), jnp.float32),
        pltpu.SemaphoreType.DMA,
    ],
)(_sc_gather_body)


_COLS = 1024
_N_TILES = BATCH // _COLS

def _tc_pack_body(a_ref, b_ref, o_ref):
    eye = (lax.broadcasted_iota(jnp.int32, (EMB_DIM, EMB_DIM), 0)
           == lax.broadcasted_iota(jnp.int32, (EMB_DIM, EMB_DIM), 1)
           ).astype(jnp.float32)
    o_ref[:, :EMB_DIM] = lax.dot_general(
        a_ref[...], eye, (((0,), (0,)), ((), ())),
        preferred_element_type=jnp.float32)
    o_ref[:, EMB_DIM:] = lax.dot_general(
        b_ref[...], eye, (((0,), (0,)), ((), ())),
        preferred_element_type=jnp.float32)


def _tc_pack(qt):
    n_steps = _HALF // _PCW
    return pl.pallas_call(
        _tc_pack_body,
        grid=(n_steps,),
        in_specs=[
            pl.BlockSpec((EMB_DIM, _PCW), lambda i: (0, i)),
            pl.BlockSpec(
                (EMB_DIM, _PCW),
                lambda i: (0, jnp.minimum(_HALF // _PCW + i, N_QUES // _PCW)),
            ),
        ],
        out_specs=pl.BlockSpec((_PCW, 2 * EMB_DIM), lambda i: (i, 0)),
        out_shape=jax.ShapeDtypeStruct((_HALF, 2 * EMB_DIM), jnp.float32),
    )(qt, qt)


def _softmax_cols(logits):
    m = jnp.max(logits, axis=0, keepdims=True)
    e = jnp.exp(logits - m)
    return e / jnp.sum(e, axis=0, keepdims=True)


def _tc_tag_body(ids_ref, tagt_ref, fwt_ref, b_ref, o_ref):
    ids = ids_ref[0]
    tgrid = lax.broadcasted_iota(jnp.int32, (N_TAG, _COLS), 0)
    onehot = (tgrid == ids).astype(jnp.float32)
    zt = lax.dot_general(tagt_ref[...], onehot, (((1,), (0,)), ((), ())),
                         preferred_element_type=jnp.float32)
    logits = lax.dot_general(fwt_ref[...], zt, (((0,), (0,)), ((), ())),
                             preferred_element_type=jnp.float32)
    o_ref[...] = _softmax_cols(logits + b_ref[...])


def _tc_ques_body(zp_ref, qid_ref, fwt_ref, b_ref, _alias_ref, o_ref):
    zp = zp_ref[...]
    hi = (qid_ref[0] - N_TAG) >= _HALF
    z = jnp.where(hi, zp[:, EMB_DIM:], zp[:, :EMB_DIM])
    logits = lax.dot_general(fwt_ref[...], z, (((0,), (1,)), ((), ())),
                             preferred_element_type=jnp.float32)
    o_ref[...] = _softmax_cols(logits + b_ref[...])


def _tc_tag(ids3, tagt, fwt, b2):
    return pl.pallas_call(
        _tc_tag_body,
        grid=(_N_TILES,),
        in_specs=[
            pl.BlockSpec((1, 1, _COLS), lambda i: (i, 0, 0)),
            pl.BlockSpec((EMB_DIM, N_TAG), lambda i: (0, 0)),
            pl.BlockSpec((EMB_DIM, N_TAG), lambda i: (0, 0)),
            pl.BlockSpec((N_TAG, 1), lambda i: (0, 0)),
        ],
        out_specs=pl.BlockSpec((N_TAG, _COLS), lambda i: (0, i)),
        out_shape=jax.ShapeDtypeStruct((N_TAG, 2 * BATCH), jnp.float32),
    )(ids3, tagt, fwt, b2)


def _tc_ques(zp, qid3, fwt, b2, out_buf):
    return pl.pallas_call(
        _tc_ques_body,
        grid=(_N_TILES,),
        in_specs=[
            pl.BlockSpec((_COLS, 2 * EMB_DIM), lambda i: (i, 0)),
            pl.BlockSpec((1, _COLS, 1), lambda i: (i, 0, 0)),
            pl.BlockSpec((EMB_DIM, N_TAG), lambda i: (0, 0)),
            pl.BlockSpec((N_TAG, 1), lambda i: (0, 0)),
            pl.BlockSpec(memory_space=pl.ANY),
        ],
        out_specs=pl.BlockSpec((N_TAG, _COLS), lambda i: (0, _N_TILES + i)),
        out_shape=jax.ShapeDtypeStruct((N_TAG, 2 * BATCH), jnp.float32),
        input_output_aliases={4: 0},
    )(zp, qid3, fwt, b2, out_buf)


def kernel(tag_ids, ques_ids, tag_table, ques_table, fc_w, fc_b):
    tag_ids = tag_ids.astype(jnp.int32)
    ques_ids = ques_ids.astype(jnp.int32)
    qcat = _tc_pack(ques_table.T)
    zp = _sc_gather(qcat, ques_ids)
    tagt = tag_table.T
    fwt = fc_w.T
    b2 = fc_b.reshape(N_TAG, 1)
    ids3 = tag_ids.reshape(_N_TILES, 1, _COLS)
    qid3 = ques_ids.reshape(_N_TILES, _COLS, 1)
    out_t = _tc_tag(ids3, tagt, fwt, b2)
    out_t = _tc_ques(zp, qid3, fwt, b2, out_t)
    return out_t.T


# speedup vs baseline: 1.4292x; 1.0246x over previous
import functools

import jax
import jax.numpy as jnp
from jax import lax
from jax.experimental import pallas as pl
from jax.experimental.pallas import tpu as pltpu
from jax.experimental.pallas import tpu_sc as plsc

N_TAG = 1000
N_QUES = 1000000
EMB_DIM = 64
BATCH = 16384

_SC_INFO = plsc.get_sparse_core_info()
_NC = _SC_INFO.num_cores
_NS = _SC_INFO.num_subcores
_NW = _NC * _NS
_BPW = BATCH // _NW


_GCH = 128


_PCW = 16384
_HALF = 31 * _PCW


def _sc_gather_body(qp_hbm, ids_hbm, out_hbm, idx_v, pidx_v, pairs_v, sem):
    wid = lax.axis_index("s") * _NC + lax.axis_index("c")
    base = wid * _BPW

    pltpu.sync_copy(ids_hbm.at[pl.ds(base, _BPW)], idx_v)

    for c in range(_BPW // _GCH):
        def _pidx(g):
            rel = idx_v[pl.ds(c * _GCH + g * 16, 16)] - N_TAG
            wrap = jnp.where(rel >= _HALF, rel - _HALF, rel)
            pidx_v[pl.ds(g * 16, 16)] = wrap

        pl.loop(0, _GCH // 16)(_pidx)

        pltpu.async_copy(qp_hbm.at[pidx_v], pairs_v, sem).wait()
        pltpu.sync_copy(pairs_v, out_hbm.at[pl.ds(base + c * _GCH, _GCH)])


_sc_gather = functools.partial(
    pl.kernel,
    mesh=plsc.VectorSubcoreMesh(core_axis_name="c", subcore_axis_name="s"),
    out_type=jax.ShapeDtypeStruct((BATCH, 2 * EMB_DIM), jnp.float32),
    scratch_types=[
        pltpu.VMEM((_BPW,), jnp.int32),
        pltpu.VMEM((_GCH,), jnp.int32),
        pltpu.VMEM((_GCH, 2 * EMB_DIM), jnp.float32),
        pltpu.SemaphoreType.DMA,
    ],
)(_sc_gather_body)


_COLS = 1024
_N_TILES = BATCH // _COLS

def _tc_pack_body(a_ref, b_ref, o_ref):
    eye = (lax.broadcasted_iota(jnp.int32, (EMB_DIM, EMB_DIM), 0)
           == lax.broadcasted_iota(jnp.int32, (EMB_DIM, EMB_DIM), 1)
           ).astype(jnp.float32)
    o_ref[:, :EMB_DIM] = lax.dot_general(
        a_ref[...], eye, (((0,), (0,)), ((), ())),
        preferred_element_type=jnp.float32)
    o_ref[:, EMB_DIM:] = lax.dot_general(
        b_ref[...], eye, (((0,), (0,)), ((), ())),
        preferred_element_type=jnp.float32)


def _tc_pack(qt):
    n_steps = _HALF // _PCW
    return pl.pallas_call(
        _tc_pack_body,
        grid=(n_steps,),
        in_specs=[
            pl.BlockSpec((EMB_DIM, _PCW), lambda i: (0, i)),
            pl.BlockSpec(
                (EMB_DIM, _PCW),
                lambda i: (0, jnp.minimum(_HALF // _PCW + i, N_QUES // _PCW)),
            ),
        ],
        out_specs=pl.BlockSpec((_PCW, 2 * EMB_DIM), lambda i: (i, 0)),
        out_shape=jax.ShapeDtypeStruct((_HALF, 2 * EMB_DIM), jnp.float32),
    )(qt, qt)


def _softmax_cols(logits):
    m = jnp.max(logits, axis=0, keepdims=True)
    e = jnp.exp(logits - m)
    return e / jnp.sum(e, axis=0, keepdims=True)


def _tc_tag_body(ids_ref, tagt_ref, fwt_ref, b_ref, o_ref):
    ids = ids_ref[0]
    tgrid = lax.broadcasted_iota(jnp.int32, (N_TAG, _COLS), 0)
    onehot = (tgrid == ids).astype(jnp.float32)
    zt = lax.dot_general(tagt_ref[...], onehot, (((1,), (0,)), ((), ())),
                         preferred_element_type=jnp.float32)
    logits = lax.dot_general(fwt_ref[...], zt, (((0,), (0,)), ((), ())),
                             preferred_element_type=jnp.float32)
    o_ref[...] = _softmax_cols(logits + b_ref[...])


def _tc_ques_body(zp_ref, qid_ref, fwt_ref, b_ref, _alias_ref, o_ref):
    zp = zp_ref[...]
    hi_l = ((qid_ref[0] - N_TAG) >= _HALF).astype(jnp.float32)
    hi = lax.transpose(hi_l, (1, 0)) == 1.0
    z = jnp.where(hi, zp[:, EMB_DIM:], zp[:, :EMB_DIM])
    logits = lax.dot_general(fwt_ref[...], z, (((0,), (1,)), ((), ())),
                             preferred_element_type=jnp.float32)
    o_ref[...] = _softmax_cols(logits + b_ref[...])


def _tc_tag(ids3, tagt, fwt, b2):
    return pl.pallas_call(
        _tc_tag_body,
        grid=(_N_TILES,),
        in_specs=[
            pl.BlockSpec((1, 1, _COLS), lambda i: (i, 0, 0)),
            pl.BlockSpec((EMB_DIM, N_TAG), lambda i: (0, 0)),
            pl.BlockSpec((EMB_DIM, N_TAG), lambda i: (0, 0)),
            pl.BlockSpec((N_TAG, 1), lambda i: (0, 0)),
        ],
        out_specs=pl.BlockSpec((N_TAG, _COLS), lambda i: (0, i)),
        out_shape=jax.ShapeDtypeStruct((N_TAG, 2 * BATCH), jnp.float32),
    )(ids3, tagt, fwt, b2)


def _tc_ques(zp, qid3, fwt, b2, out_buf):
    return pl.pallas_call(
        _tc_ques_body,
        grid=(_N_TILES,),
        in_specs=[
            pl.BlockSpec((_COLS, 2 * EMB_DIM), lambda i: (i, 0)),
            pl.BlockSpec((1, 1, _COLS), lambda i: (i, 0, 0)),
            pl.BlockSpec((EMB_DIM, N_TAG), lambda i: (0, 0)),
            pl.BlockSpec((N_TAG, 1), lambda i: (0, 0)),
            pl.BlockSpec(memory_space=pl.ANY),
        ],
        out_specs=pl.BlockSpec((N_TAG, _COLS), lambda i: (0, _N_TILES + i)),
        out_shape=jax.ShapeDtypeStruct((N_TAG, 2 * BATCH), jnp.float32),
        input_output_aliases={4: 0},
    )(zp, qid3, fwt, b2, out_buf)


def kernel(tag_ids, ques_ids, tag_table, ques_table, fc_w, fc_b):
    tag_ids = tag_ids.astype(jnp.int32)
    ques_ids = ques_ids.astype(jnp.int32)
    qcat = _tc_pack(ques_table.T)
    zp = _sc_gather(qcat, ques_ids)
    tagt = tag_table.T
    fwt = fc_w.T
    b2 = fc_b.reshape(N_TAG, 1)
    ids3 = tag_ids.reshape(_N_TILES, 1, _COLS)
    qid3 = ques_ids.reshape(_N_TILES, 1, _COLS)
    out_t = _tc_tag(ids3, tagt, fwt, b2)
    out_t = _tc_ques(zp, qid3, fwt, b2, out_t)
    return out_t.T
